# baseline (speedup 1.0000x reference)
import functools

import jax
import jax.numpy as jnp
from jax import lax
from jax.experimental import pallas as pl
from jax.experimental.pallas import tpu as pltpu
from jax.experimental.pallas import tpu_sc as plsc

_B, _N = 2, 6
_C_IN, _C_T, _D = 512, 64, 59
_FH, _FW = 16, 44
_NPIX = _FH * _FW
_BN = _B * _N
_G = 128
_PIXB = _N * _NPIX
_ROWS_PER_B = _G * _G
_WROWS = 33024
_TRASH = 2 * _ROWS_PER_B * _PIXB
_WSIZE = _WROWS * _PIXB

_NSC, _NSUB = 2, 16
_PTS_PER_B = _N * _D * _NPIX
_PPW = _PTS_PER_B // _NSUB
_CH = 128
_NCHUNK = -(-_PPW // _CH)
_PPWP = _NCHUNK * _CH
_PERSUB = _ROWS_PER_B * _PIXB // _NSUB
_ZBUF = 16384


def _stage_a_body(x_ref, w_ref, b_ref, key_ref, idx_ref, val_ref, imf_ref):
    bn = pl.program_id(0)
    feat = jnp.dot(w_ref[...], x_ref[0], preferred_element_type=jnp.float32,
                   precision=lax.Precision.HIGHEST) + b_ref[...]
    logits = feat[0:_D, :]
    m = jnp.max(logits, axis=0, keepdims=True)
    e = jnp.exp(logits - m)
    depth = e / jnp.sum(e, axis=0, keepdims=True)
    imf_ref[0] = feat[_D:_D + _C_T, :].T

    pixi = lax.broadcasted_iota(jnp.int32, (_D, _NPIX), 1)
    key = key_ref[0]
    kept = key >= 0
    cell = key

    flag = jnp.concatenate(
        [jnp.zeros((1, _NPIX), jnp.float32),
         (key[1:] == key[:-1]).astype(jnp.float32)], 0)
    acc = depth
    for sh in (1, 2, 4, 8, 16, 32):
        acc_sh = jnp.concatenate(
            [jnp.zeros((sh, _NPIX), jnp.float32), acc[:-sh]], 0)
        flag_sh = jnp.concatenate(
            [jnp.zeros((sh, _NPIX), jnp.float32), flag[:-sh]], 0)
        acc = acc + flag * acc_sh
        flag = flag * flag_sh
    end_f = jnp.concatenate(
        [(key[:-1] != key[1:]).astype(jnp.float32),
         jnp.ones((1, _NPIX), jnp.float32)], 0)
    emit = (end_f > 0.0) & kept

    b = bn // _N
    n = bn % _N
    col = n * _NPIX + pixi
    row = b * _ROWS_PER_B + cell
    val_ref[0] = jnp.where(emit, acc, 0.0)
    idx_ref[0] = jnp.where(emit, row * _PIXB + col, _TRASH + col)


def _stage_a(x2, w_pad, b_pad, key, interpret=False):
    return pl.pallas_call(
        _stage_a_body,
        grid=(_BN,),
        in_specs=[
            pl.BlockSpec((1, _C_IN, _NPIX), lambda i: (i, 0, 0)),
            pl.BlockSpec((128, _C_IN), lambda i: (0, 0)),
            pl.BlockSpec((128, 1), lambda i: (0, 0)),
            pl.BlockSpec((1, _D, _NPIX), lambda i: (i, 0, 0)),
        ],
        out_specs=[
            pl.BlockSpec((1, _D, _NPIX), lambda i: (i, 0, 0)),
            pl.BlockSpec((1, _D, _NPIX), lambda i: (i, 0, 0)),
            pl.BlockSpec((1, _NPIX, _C_T), lambda i: (i, 0, 0)),
        ],
        out_shape=[
            jax.ShapeDtypeStruct((_BN, _D, _NPIX), jnp.int32),
            jax.ShapeDtypeStruct((_BN, _D, _NPIX), jnp.float32),
            jax.ShapeDtypeStruct((_BN, _NPIX, _C_T), jnp.float32),
        ],
        interpret=interpret,
    )(x2, w_pad, b_pad, key)


def _sc_scatter_body(idx_hbm, val_hbm, w_hbm, zbuf, idxb, valb, zsem, ssem):
    c = lax.axis_index("c")
    s = lax.axis_index("s")

    def zinit(i, carry):
        zbuf[pl.ds(i * 16, 16)] = jnp.zeros((16,), jnp.float32)
        return carry
    lax.fori_loop(0, _ZBUF // 16, zinit, 0)

    base = c * (_ROWS_PER_B * _PIXB) + s * _PERSUB

    def zdma(i, carry):
        cps = [pltpu.async_copy(
            zbuf, w_hbm.at[pl.ds(base + (i * 4 + k) * _ZBUF, _ZBUF)], zsem)
            for k in range(4)]
        for cp in cps:
            cp.wait()
        return carry
    lax.fori_loop(0, _PERSUB // _ZBUF // 4, zdma, 0)

    plsc.subcore_barrier()

    pltpu.sync_copy(idx_hbm.at[c, s], idxb)
    pltpu.sync_copy(val_hbm.at[c, s], valb)

    def sc(j, carry):
        pltpu.async_copy(valb.at[j], w_hbm.at[idxb.at[j]], ssem).wait()
        return carry
    lax.fori_loop(0, _NCHUNK, sc, 0)


@functools.cache
def _make_sc_scatter():
    return pl.kernel(
        _sc_scatter_body,
        out_type=jax.ShapeDtypeStruct((_WSIZE,), jnp.float32),
        mesh=plsc.VectorSubcoreMesh(core_axis_name="c", subcore_axis_name="s"),
        scratch_types=[
            pltpu.VMEM((_ZBUF,), jnp.float32),
            pltpu.VMEM((_NCHUNK, _CH), jnp.int32),
            pltpu.VMEM((_NCHUNK, _CH), jnp.float32),
            pltpu.SemaphoreType.DMA,
            pltpu.SemaphoreType.DMA,
        ],
    )


def _stage_c_body(w_ref, imf_ref, out_ref):
    out_ref[0] = jnp.dot(w_ref[...], imf_ref[0],
                         preferred_element_type=jnp.float32,
                         precision=lax.Precision.HIGHEST)


def _stage_c(w2, imf_b, interpret=False):
    return pl.pallas_call(
        _stage_c_body,
        grid=(_B, _ROWS_PER_B // 256),
        in_specs=[
            pl.BlockSpec((256, _PIXB), lambda b, m: (b * 64 + m, 0)),
            pl.BlockSpec((1, _PIXB, _C_T), lambda b, m: (b, 0, 0)),
        ],
        out_specs=pl.BlockSpec((1, 256, _C_T), lambda b, m: (b, m, 0)),
        out_shape=jax.ShapeDtypeStruct((_B, _ROWS_PER_B, _C_T), jnp.float32),
        interpret=interpret,
    )(w2, imf_b)


def _voxel_key(rots, trans, intrins, post_rots, post_trans):
    ds = (jnp.arange(1.0, 60.0, 1.0, dtype=jnp.float32).reshape(_D, 1, 1)
          * jnp.ones((_D, _FH, _FW), jnp.float32))
    xs = (jnp.linspace(0.0, 704 - 1.0, _FW, dtype=jnp.float32)
          .reshape(1, 1, _FW) * jnp.ones((_D, _FH, _FW), jnp.float32))
    ys = (jnp.linspace(0.0, 256 - 1.0, _FH, dtype=jnp.float32)
          .reshape(1, _FH, 1) * jnp.ones((_D, _FH, _FW), jnp.float32))
    frustum = jnp.stack((xs, ys, ds), -1)
    pts = frustum[None, None] - post_trans[:, :, None, None, None, :]
    inv_pr = jnp.linalg.inv(post_rots)
    pts = jnp.einsum('bnij,bndhwj->bndhwi', inv_pr, pts)
    pts = jnp.concatenate([pts[..., :2] * pts[..., 2:3], pts[..., 2:3]], -1)
    combine = rots @ jnp.linalg.inv(intrins)
    pts = (jnp.einsum('bnij,bndhwj->bndhwi', combine, pts)
           + trans[:, :, None, None, None, :])
    dx = jnp.array([0.8, 0.8, 20.0], dtype=jnp.float32)
    bx = jnp.array([-50.8, -50.8, 0.0], dtype=jnp.float32)
    g = ((pts - (bx - dx / 2.0)) / dx).astype(jnp.int32)
    kept = ((g[..., 0] >= 0) & (g[..., 0] < _G) & (g[..., 1] >= 0)
            & (g[..., 1] < _G) & (g[..., 2] >= 0) & (g[..., 2] < 1))
    cell = g[..., 0] * _G + g[..., 1]
    return jnp.where(kept, cell, -1).reshape(_BN, _D, _NPIX)


def kernel(x, rots, trans, intrins, post_rots, post_trans, W_depth, b_depth):
    x2 = x.reshape(_BN, _C_IN, _NPIX)
    w_pad = jnp.zeros((128, _C_IN), jnp.float32).at[:_D + _C_T].set(W_depth)
    b_pad = jnp.zeros((128, 1), jnp.float32).at[:_D + _C_T, 0].set(b_depth)
    key = _voxel_key(rots, trans, intrins, post_rots, post_trans)

    idx, val, imf = _stage_a(x2, w_pad, b_pad, key)

    idx2 = idx.reshape(_B, _PTS_PER_B)
    val2 = val.reshape(_B, _PTS_PER_B)
    pad = _NSUB * _PPWP - _PTS_PER_B
    idx4 = jnp.pad(idx2, ((0, 0), (0, pad)), constant_values=_TRASH)
    idx4 = idx4.reshape(_B, _NSUB, _NCHUNK, _CH)
    val4 = jnp.pad(val2, ((0, 0), (0, pad))).reshape(_B, _NSUB, _NCHUNK, _CH)

    w_flat = _make_sc_scatter()(idx4, val4)
    w2 = w_flat.reshape(_WROWS, _PIXB)

    imf_b = imf.reshape(_B, _PIXB, _C_T)
    out = _stage_c(w2, imf_b)
    return (out.reshape(_B, _G, _G, _C_T).transpose(0, 3, 1, 2)
            .reshape(_B, _C_T, _G, _G))


# speedup vs baseline: 1.6631x; 1.6631x over previous
import functools

import jax
import jax.numpy as jnp
from jax import lax
from jax.experimental import pallas as pl
from jax.experimental.pallas import tpu as pltpu
from jax.experimental.pallas import tpu_sc as plsc

_B, _N = 2, 6
_C_IN, _C_T, _D = 512, 64, 59
_FH, _FW = 16, 44
_NPIX = _FH * _FW
_BN = _B * _N
_G = 128
_PIXB = _N * _NPIX
_ROWS_PER_B = _G * _G
_WROWS = 33024
_TRASH = 2 * _ROWS_PER_B * _PIXB
_WSIZE = _WROWS * _PIXB

_NSC, _NSUB = 2, 16
_PTS_PER_B = _N * _D * _NPIX
_PPW = _PTS_PER_B // _NSUB
_CH = 128
_NCHUNK = -(-_PPW // _CH)
_PPWP = _NCHUNK * _CH
_PERSUB = _ROWS_PER_B * _PIXB // _NSUB
_ZBUF = 16384


def _stage_a_body(x_ref, w_ref, b_ref, key_ref, idx_ref, val_ref, imf_ref):
    bn = pl.program_id(0)
    feat = jnp.dot(w_ref[...], x_ref[0], preferred_element_type=jnp.float32,
                   precision=lax.Precision.HIGHEST) + b_ref[...]
    logits = feat[0:_D, :]
    m = jnp.max(logits, axis=0, keepdims=True)
    e = jnp.exp(logits - m)
    depth = e / jnp.sum(e, axis=0, keepdims=True)
    imf_ref[0] = feat[_D:_D + _C_T, :].T

    pixi = lax.broadcasted_iota(jnp.int32, (_D, _NPIX), 1)
    key = key_ref[0]
    kept = key >= 0
    cell = key

    flag = jnp.concatenate(
        [jnp.zeros((1, _NPIX), jnp.float32),
         (key[1:] == key[:-1]).astype(jnp.float32)], 0)
    acc = depth
    for sh in (1, 2, 4, 8, 16, 32):
        acc_sh = jnp.concatenate(
            [jnp.zeros((sh, _NPIX), jnp.float32), acc[:-sh]], 0)
        flag_sh = jnp.concatenate(
            [jnp.zeros((sh, _NPIX), jnp.float32), flag[:-sh]], 0)
        acc = acc + flag * acc_sh
        flag = flag * flag_sh
    end_f = jnp.concatenate(
        [(key[:-1] != key[1:]).astype(jnp.float32),
         jnp.ones((1, _NPIX), jnp.float32)], 0)
    emit = (end_f > 0.0) & kept

    b = bn // _N
    n = bn % _N
    col = n * _NPIX + pixi
    row = b * _ROWS_PER_B + cell
    di = lax.broadcasted_iota(jnp.int32, (_D, _NPIX), 0)
    p_global = (bn * _D + di) * _NPIX + pixi
    val_ref[0] = jnp.where(emit, acc, 0.0)
    idx_ref[0] = jnp.where(emit, row * _PIXB + col, _TRASH + p_global)


def _stage_a(x2, w_pad, b_pad, key, interpret=False):
    return pl.pallas_call(
        _stage_a_body,
        grid=(_BN,),
        in_specs=[
            pl.BlockSpec((1, _C_IN, _NPIX), lambda i: (i, 0, 0)),
            pl.BlockSpec((128, _C_IN), lambda i: (0, 0)),
            pl.BlockSpec((128, 1), lambda i: (0, 0)),
            pl.BlockSpec((1, _D, _NPIX), lambda i: (i, 0, 0)),
        ],
        out_specs=[
            pl.BlockSpec((1, _D, _NPIX), lambda i: (i, 0, 0)),
            pl.BlockSpec((1, _D, _NPIX), lambda i: (i, 0, 0)),
            pl.BlockSpec((1, _NPIX, _C_T), lambda i: (i, 0, 0)),
        ],
        out_shape=[
            jax.ShapeDtypeStruct((_BN, _D, _NPIX), jnp.int32),
            jax.ShapeDtypeStruct((_BN, _D, _NPIX), jnp.float32),
            jax.ShapeDtypeStruct((_BN, _NPIX, _C_T), jnp.float32),
        ],
        interpret=interpret,
    )(x2, w_pad, b_pad, key)


def _sc_scatter_body(idx_hbm, val_hbm, w_hbm, zbuf, idxb, valb, zsem, ssem):
    c = lax.axis_index("c")
    s = lax.axis_index("s")

    def zinit(i, carry):
        zbuf[pl.ds(i * 16, 16)] = jnp.zeros((16,), jnp.float32)
        return carry
    lax.fori_loop(0, _ZBUF // 16, zinit, 0)

    base = c * (_ROWS_PER_B * _PIXB) + s * _PERSUB

    def zdma(i, carry):
        cps = [pltpu.async_copy(
            zbuf, w_hbm.at[pl.ds(base + (i * 4 + k) * _ZBUF, _ZBUF)], zsem)
            for k in range(4)]
        for cp in cps:
            cp.wait()
        return carry
    lax.fori_loop(0, _PERSUB // _ZBUF // 4, zdma, 0)

    plsc.subcore_barrier()

    pltpu.sync_copy(idx_hbm.at[c, s], idxb)
    pltpu.sync_copy(val_hbm.at[c, s], valb)

    def sc(i, carry):
        cps = [pltpu.async_copy(
            valb.at[i * 2 + k], w_hbm.at[idxb.at[i * 2 + k]], ssem)
            for k in range(2)]
        for cp in cps:
            cp.wait()
        return carry
    lax.fori_loop(0, _NCHUNK // 2, sc, 0)


@functools.cache
def _make_sc_scatter():
    return pl.kernel(
        _sc_scatter_body,
        out_type=jax.ShapeDtypeStruct((_WSIZE,), jnp.float32),
        mesh=plsc.VectorSubcoreMesh(core_axis_name="c", subcore_axis_name="s"),
        scratch_types=[
            pltpu.VMEM((_ZBUF,), jnp.float32),
            pltpu.VMEM((_NCHUNK, _CH), jnp.int32),
            pltpu.VMEM((_NCHUNK, _CH), jnp.float32),
            pltpu.SemaphoreType.DMA,
            pltpu.SemaphoreType.DMA,
        ],
    )


def _stage_c_body(w_ref, imf_ref, out_ref):
    out_ref[0] = jnp.dot(w_ref[...], imf_ref[0],
                         preferred_element_type=jnp.float32)


def _stage_c(w2, imf_b, interpret=False):
    return pl.pallas_call(
        _stage_c_body,
        grid=(_B, _ROWS_PER_B // 256),
        in_specs=[
            pl.BlockSpec((256, _PIXB), lambda b, m: (b * 64 + m, 0)),
            pl.BlockSpec((1, _PIXB, _C_T), lambda b, m: (b, 0, 0)),
        ],
        out_specs=pl.BlockSpec((1, 256, _C_T), lambda b, m: (b, m, 0)),
        out_shape=jax.ShapeDtypeStruct((_B, _ROWS_PER_B, _C_T), jnp.float32),
        interpret=interpret,
    )(w2, imf_b)


def _voxel_key(rots, trans, intrins, post_rots, post_trans):
    ds = (jnp.arange(1.0, 60.0, 1.0, dtype=jnp.float32).reshape(_D, 1, 1)
          * jnp.ones((_D, _FH, _FW), jnp.float32))
    xs = (jnp.linspace(0.0, 704 - 1.0, _FW, dtype=jnp.float32)
          .reshape(1, 1, _FW) * jnp.ones((_D, _FH, _FW), jnp.float32))
    ys = (jnp.linspace(0.0, 256 - 1.0, _FH, dtype=jnp.float32)
          .reshape(1, _FH, 1) * jnp.ones((_D, _FH, _FW), jnp.float32))
    frustum = jnp.stack((xs, ys, ds), -1)
    pts = frustum[None, None] - post_trans[:, :, None, None, None, :]
    inv_pr = jnp.linalg.inv(post_rots)
    pts = jnp.einsum('bnij,bndhwj->bndhwi', inv_pr, pts)
    pts = jnp.concatenate([pts[..., :2] * pts[..., 2:3], pts[..., 2:3]], -1)
    combine = rots @ jnp.linalg.inv(intrins)
    pts = (jnp.einsum('bnij,bndhwj->bndhwi', combine, pts)
           + trans[:, :, None, None, None, :])
    dx = jnp.array([0.8, 0.8, 20.0], dtype=jnp.float32)
    bx = jnp.array([-50.8, -50.8, 0.0], dtype=jnp.float32)
    g = ((pts - (bx - dx / 2.0)) / dx).astype(jnp.int32)
    kept = ((g[..., 0] >= 0) & (g[..., 0] < _G) & (g[..., 1] >= 0)
            & (g[..., 1] < _G) & (g[..., 2] >= 0) & (g[..., 2] < 1))
    cell = g[..., 0] * _G + g[..., 1]
    return jnp.where(kept, cell, -1).reshape(_BN, _D, _NPIX)


def kernel(x, rots, trans, intrins, post_rots, post_trans, W_depth, b_depth):
    x2 = x.reshape(_BN, _C_IN, _NPIX)
    w_pad = jnp.zeros((128, _C_IN), jnp.float32).at[:_D + _C_T].set(W_depth)
    b_pad = jnp.zeros((128, 1), jnp.float32).at[:_D + _C_T, 0].set(b_depth)
    key = _voxel_key(rots, trans, intrins, post_rots, post_trans)

    idx, val, imf = _stage_a(x2, w_pad, b_pad, key)

    idx2 = idx.reshape(_B, _PTS_PER_B)
    val2 = val.reshape(_B, _PTS_PER_B)
    pad = _NSUB * _PPWP - _PTS_PER_B
    pad_idx = (_TRASH + _B * _PTS_PER_B
               + jnp.arange(_B * pad, dtype=jnp.int32).reshape(_B, pad))
    idx4 = jnp.concatenate([idx2, pad_idx], axis=1)
    idx4 = idx4.reshape(_B, _NSUB, _NCHUNK, _CH)
    val4 = jnp.pad(val2, ((0, 0), (0, pad))).reshape(_B, _NSUB, _NCHUNK, _CH)

    w_flat = _make_sc_scatter()(idx4, val4)
    w2 = w_flat.reshape(_WROWS, _PIXB)

    imf_b = imf.reshape(_B, _PIXB, _C_T)
    out = _stage_c(w2, imf_b)
    return (out.reshape(_B, _G, _G, _C_T).transpose(0, 3, 1, 2)
            .reshape(_B, _C_T, _G, _G))
